# Initial kernel scaffold; baseline (speedup 1.0000x reference)
import jax
import jax.numpy as jnp
from jax.experimental import pallas as pl


def kernel(x1, edge_index1, edge_attr1, batch1, x2, edge_index2, edge_attr2, batch2, W1, b1, W2, b2, fcW, fcb):
    raise NotImplementedError("write your pallas kernel here")


# speedup vs baseline: 9.3773x; 9.3773x over previous
import functools

import jax
import jax.numpy as jnp
from jax.experimental import pallas as pl
from jax.experimental.pallas import tpu as pltpu

_N = 10000
_G = 64
_BLK = 2000


def _fused_body(nblk, x1_ref, bat1_ref, x2_ref, bat2_ref, w1_ref, b1_ref,
                w2_ref, b2_ref, fcw_ref, fcb_ref, out_ref,
                s1_ref, c1_ref, s2_ref, c2_ref):
    i = pl.program_id(0)

    @pl.when(i == 0)
    def _init():
        s1_ref[...] = jnp.zeros_like(s1_ref)
        c1_ref[...] = jnp.zeros_like(c1_ref)
        s2_ref[...] = jnp.zeros_like(s2_ref)
        c2_ref[...] = jnp.zeros_like(c2_ref)

    gids = jax.lax.broadcasted_iota(jnp.int32, (1, _G), 1)
    ones_col = jnp.ones((x1_ref.shape[0], 1), dtype=jnp.float32)

    def accum(x_ref, bat_ref, w_ref, b_ref, s_ref, c_ref):
        h = jnp.maximum(
            jnp.dot(x_ref[...], w_ref[...],
                    preferred_element_type=jnp.float32) + b_ref[...], 0.0)
        onehot = (bat_ref[...] == gids).astype(jnp.float32)
        s_ref[...] += jax.lax.dot_general(
            onehot, h, (((0,), (0,)), ((), ())),
            preferred_element_type=jnp.float32)
        c_ref[...] += jax.lax.dot_general(
            onehot, ones_col, (((0,), (0,)), ((), ())),
            preferred_element_type=jnp.float32)

    accum(x1_ref, bat1_ref, w1_ref, b1_ref, s1_ref, c1_ref)
    accum(x2_ref, bat2_ref, w2_ref, b2_ref, s2_ref, c2_ref)

    @pl.when(i == nblk - 1)
    def _finish():
        p1 = s1_ref[...] / jnp.maximum(c1_ref[...], 1.0)
        p2 = s2_ref[...] / jnp.maximum(c2_ref[...], 1.0)
        pool = (p1 + p2) * 0.5
        out_ref[...] = jnp.dot(pool, fcw_ref[...],
                               preferred_element_type=jnp.float32) + fcb_ref[...]


@jax.jit
def _run(x1, bat1, x2, bat2, W1, b1, W2, b2, fcW, fcb):
    n, f1 = x1.shape
    h = W1.shape[1]
    out_dim = fcW.shape[1]
    nblk = n // _BLK

    row_spec = pl.BlockSpec((_BLK, f1), lambda i: (i, 0))
    bat_spec = pl.BlockSpec((_BLK, 1), lambda i: (i, 0))
    full = lambda a: pl.BlockSpec(a.shape, lambda i: (0,) * a.ndim)

    return pl.pallas_call(
        functools.partial(_fused_body, nblk),
        grid=(nblk,),
        in_specs=[row_spec, bat_spec, row_spec, bat_spec,
                  full(W1), full(b1), full(W2), full(b2),
                  full(fcW), full(fcb)],
        out_specs=pl.BlockSpec((_G, out_dim), lambda i: (0, 0)),
        out_shape=jax.ShapeDtypeStruct((_G, out_dim), jnp.float32),
        scratch_shapes=[
            pltpu.VMEM((_G, h), jnp.float32),
            pltpu.VMEM((_G, 1), jnp.float32),
            pltpu.VMEM((_G, h), jnp.float32),
            pltpu.VMEM((_G, 1), jnp.float32),
        ],
    )(x1, bat1, x2, bat2, W1, b1, W2, b2, fcW, fcb)


def kernel(x1, edge_index1, edge_attr1, batch1, x2, edge_index2, edge_attr2,
           batch2, W1, b1, W2, b2, fcW, fcb):
    del edge_index1, edge_attr1, edge_index2, edge_attr2
    return _run(x1, batch1.reshape(-1, 1), x2, batch2.reshape(-1, 1),
                W1, b1.reshape(1, -1), W2, b2.reshape(1, -1),
                fcW, fcb.reshape(1, -1))
